# baseline (speedup 1.0000x reference)
import functools

import jax
import jax.numpy as jnp
from jax import lax
from jax.experimental import pallas as pl
from jax.experimental.pallas import tpu as pltpu
from jax.experimental.pallas import tpu_sc as plsc

N_NODES = 100000
N_EDGES = 100000
N_CONN = 6400000
W_MAX = 1.0

NC = 2
NS = 16
LANES = 16

NW = NC * NS
PER_W = N_CONN // NW
CHUNK = 2000
NCH = PER_W // CHUNK
UNROLL = 5
TPAD = 102400
TSLICE = TPAD // NS
PIECE = 1600
GSL = TPAD // NW

_vmesh = plsc.VectorSubcoreMesh(core_axis_name="c", subcore_axis_name="s")
_sc_params = pltpu.CompilerParams(needs_layout_passes=False)


def _fill(ref, n, value):
    def body(i, _):
        ref[pl.ds(i * LANES, LANES)] = jnp.full((LANES,), value, jnp.float32)
        return 0
    lax.fori_loop(0, n // LANES, body, 0)


def _gather_chunk(table_v, ni, vv):
    def body(i, _):
        for k in range(UNROLL):
            dsl = pl.ds((UNROLL * i + k) * LANES, LANES)
            vv[dsl] = plsc.load_gather(table_v, [ni[dsl]])
        return 0
    lax.fori_loop(0, CHUNK // (UNROLL * LANES), body, 0)


@functools.partial(
    pl.kernel,
    mesh=_vmesh,
    out_type=[
        jax.ShapeDtypeStruct((NC * TPAD,), jnp.float32),
        jax.ShapeDtypeStruct((NC * TPAD,), jnp.float32),
    ],
    compiler_params=_sc_params,
    scratch_types=[
        pltpu.VMEM((N_NODES,), jnp.float32),
        pltpu.VMEM((CHUNK,), jnp.int32),
        pltpu.VMEM((CHUNK,), jnp.int32),
        pltpu.VMEM((CHUNK,), jnp.int32),
        pltpu.VMEM((CHUNK,), jnp.int32),
        pltpu.VMEM((CHUNK,), jnp.float32),
        pltpu.VMEM((CHUNK,), jnp.float32),
        pltpu.VMEM((CHUNK,), jnp.float32),
        pltpu.VMEM_SHARED((TPAD,), jnp.float32),
        pltpu.VMEM_SHARED((TPAD,), jnp.float32),
        pltpu.SemaphoreType.DMA,
        pltpu.SemaphoreType.DMA,
    ],
)
def _edge_accum(x_hbm, he_hbm, s_out, c_out, x_v, ni0, ni1, ei0, ei1, v0, v1,
                ones_v, s_sp, c_sp, sem0, sem1):
    cid = lax.axis_index("c")
    sid = lax.axis_index("s")
    wid = sid * NC + cid
    tlo = sid * TSLICE

    _fill(v0, CHUNK, 0.0)
    for k in range(TSLICE // PIECE):
        piece = pl.ds(tlo + k * PIECE, PIECE)
        pltpu.sync_copy(v0.at[pl.ds(0, PIECE)], s_sp.at[piece])
        pltpu.sync_copy(v0.at[pl.ds(0, PIECE)], c_sp.at[piece])
    _fill(ones_v, CHUNK, 1.0)
    pltpu.sync_copy(x_hbm, x_v)
    plsc.subcore_barrier()

    def load_idx(chunk, ni, ei):
        base = wid * PER_W + chunk * CHUNK
        pltpu.sync_copy(he_hbm.at[pl.ds(base, CHUNK)], ni)
        pltpu.sync_copy(he_hbm.at[pl.ds(N_CONN + base, CHUNK)], ei)

    load_idx(0, ni0, ei0)
    _gather_chunk(x_v, ni0, v0)

    def pipe_body(j, _):
        c0 = 2 * j
        s0 = pltpu.async_copy(v0, s_sp.at[ei0], sem0, add=True)
        c0d = pltpu.async_copy(ones_v, c_sp.at[ei0], sem0, add=True)
        load_idx(c0 + 1, ni1, ei1)
        _gather_chunk(x_v, ni1, v1)
        s1 = pltpu.async_copy(v1, s_sp.at[ei1], sem1, add=True)
        c1d = pltpu.async_copy(ones_v, c_sp.at[ei1], sem1, add=True)
        s0.wait()
        c0d.wait()
        load_idx(jnp.minimum(c0 + 2, NCH - 1), ni0, ei0)
        _gather_chunk(x_v, ni0, v0)
        s1.wait()
        c1d.wait()
        return 0

    lax.fori_loop(0, NCH // 2, pipe_body, 0)
    plsc.subcore_barrier()

    for k in range(TSLICE // PIECE):
        src_s = pl.ds(tlo + k * PIECE, PIECE)
        dst_s = pl.ds(cid * TPAD + tlo + k * PIECE, PIECE)
        pltpu.sync_copy(s_sp.at[src_s], v0.at[pl.ds(0, PIECE)])
        pltpu.sync_copy(v0.at[pl.ds(0, PIECE)], s_out.at[dst_s])
        pltpu.sync_copy(c_sp.at[src_s], v1.at[pl.ds(0, PIECE)])
        pltpu.sync_copy(v1.at[pl.ds(0, PIECE)], c_out.at[dst_s])


@functools.partial(
    pl.kernel,
    mesh=_vmesh,
    out_type=jax.ShapeDtypeStruct((TPAD,), jnp.float32),
    compiler_params=_sc_params,
    scratch_types=[
        pltpu.VMEM((GSL,), jnp.float32),
        pltpu.VMEM((GSL,), jnp.float32),
        pltpu.VMEM((GSL,), jnp.float32),
        pltpu.VMEM((GSL,), jnp.float32),
        pltpu.VMEM((GSL,), jnp.float32),
        pltpu.VMEM((GSL,), jnp.float32),
    ],
)
def _edge_gain(s_hbm, c_hbm, wh_hbm, se_hbm, g_out, sa, sb, ca, cb, wh_v,
               se_v):
    cid = lax.axis_index("c")
    sid = lax.axis_index("s")
    wid = sid * NC + cid
    lo = wid * GSL

    pltpu.sync_copy(s_hbm.at[pl.ds(lo, GSL)], sa)
    pltpu.sync_copy(s_hbm.at[pl.ds(TPAD + lo, GSL)], sb)
    pltpu.sync_copy(c_hbm.at[pl.ds(lo, GSL)], ca)
    pltpu.sync_copy(c_hbm.at[pl.ds(TPAD + lo, GSL)], cb)
    pltpu.sync_copy(wh_hbm.at[pl.ds(lo, GSL)], wh_v)
    pltpu.sync_copy(se_hbm.at[pl.ds(lo, GSL)], se_v)

    def g_body(i, _):
        dsl = pl.ds(i * LANES, LANES)
        s = sa[dsl] + sb[dsl]
        c = jnp.maximum(ca[dsl] + cb[dsl], 1.0)
        w = wh_v[dsl] * se_v[dsl]
        se_v[dsl] = w * w * s / c
        return 0

    lax.fori_loop(0, GSL // LANES, g_body, 0)
    pltpu.sync_copy(se_v, g_out.at[pl.ds(lo, GSL)])


@functools.partial(
    pl.kernel,
    mesh=_vmesh,
    out_type=jax.ShapeDtypeStruct((NC * TPAD,), jnp.float32),
    compiler_params=_sc_params,
    scratch_types=[
        pltpu.VMEM((N_EDGES,), jnp.float32),
        pltpu.VMEM((CHUNK,), jnp.int32),
        pltpu.VMEM((CHUNK,), jnp.int32),
        pltpu.VMEM((CHUNK,), jnp.int32),
        pltpu.VMEM((CHUNK,), jnp.int32),
        pltpu.VMEM((CHUNK,), jnp.float32),
        pltpu.VMEM((CHUNK,), jnp.float32),
        pltpu.VMEM((TSLICE,), jnp.float32),
        pltpu.VMEM_SHARED((TPAD,), jnp.float32),
        pltpu.SemaphoreType.DMA,
        pltpu.SemaphoreType.DMA,
    ],
)
def _node_scatter(he_hbm, g_hbm, out_p, g_v, ni0, ni1, ei0, ei1, v0, v1,
                  stage_v, o_sp, sem0, sem1):
    cid = lax.axis_index("c")
    sid = lax.axis_index("s")
    wid = sid * NC + cid
    tlo = sid * TSLICE

    _fill(stage_v, TSLICE, 0.0)
    pltpu.sync_copy(stage_v, o_sp.at[pl.ds(tlo, TSLICE)])
    pltpu.sync_copy(g_hbm.at[pl.ds(0, N_EDGES)], g_v)
    plsc.subcore_barrier()

    def load_idx(chunk, ni, ei):
        base = wid * PER_W + chunk * CHUNK
        pltpu.sync_copy(he_hbm.at[pl.ds(base, CHUNK)], ni)
        pltpu.sync_copy(he_hbm.at[pl.ds(N_CONN + base, CHUNK)], ei)

    load_idx(0, ni0, ei0)
    _gather_chunk(g_v, ei0, v0)

    def pipe_body(j, _):
        c0 = 2 * j
        s0 = pltpu.async_copy(v0, o_sp.at[ni0], sem0, add=True)
        load_idx(c0 + 1, ni1, ei1)
        _gather_chunk(g_v, ei1, v1)
        s1 = pltpu.async_copy(v1, o_sp.at[ni1], sem1, add=True)
        s0.wait()
        load_idx(jnp.minimum(c0 + 2, NCH - 1), ni0, ei0)
        _gather_chunk(g_v, ei0, v0)
        s1.wait()
        return 0

    lax.fori_loop(0, NCH // 2, pipe_body, 0)
    plsc.subcore_barrier()

    pltpu.sync_copy(o_sp.at[pl.ds(tlo, TSLICE)], stage_v)
    pltpu.sync_copy(stage_v, out_p.at[pl.ds(cid * TPAD + tlo, TSLICE)])


def _combine_body(p_ref, wn_ref, b_ref, o_ref):
    o_ref[...] = ((p_ref[0:1, :] + p_ref[1:2, :])
                  * (W_MAX * W_MAX * wn_ref[0, 0]) + b_ref[0, 0])


def kernel(x_in, hyperedge_index, weight_node, bias, w_hat, s_e):
    wh_pad = jnp.zeros((TPAD,), jnp.float32).at[:N_EDGES].set(w_hat)
    se_pad = jnp.zeros((TPAD,), jnp.float32).at[:N_EDGES].set(s_e)
    he_flat = hyperedge_index.astype(jnp.int32).reshape(2 * N_CONN)

    s_part, c_part = _edge_accum(x_in[:, 0], he_flat)
    g = _edge_gain(s_part, c_part, wh_pad, se_pad)
    out_part = _node_scatter(he_flat, g)

    out2 = pl.pallas_call(
        _combine_body,
        out_shape=jax.ShapeDtypeStruct((1, TPAD), jnp.float32),
    )(out_part.reshape(NC, TPAD), weight_node, bias.reshape(1, 1))
    return out2[0, :N_NODES, None]


# speedup vs baseline: 1.0248x; 1.0248x over previous
import functools

import jax
import jax.numpy as jnp
from jax import lax
from jax.experimental import pallas as pl
from jax.experimental.pallas import tpu as pltpu
from jax.experimental.pallas import tpu_sc as plsc

N_NODES = 100000
N_EDGES = 100000
N_CONN = 6400000
W_MAX = 1.0

NC = 2
NS = 16
LANES = 16

NW = NC * NS
PER_W = N_CONN // NW
CHUNK = 10000
NCH = PER_W // CHUNK
TPAD = 102400
TSLICE = TPAD // NS

_vmesh = plsc.VectorSubcoreMesh(core_axis_name="c", subcore_axis_name="s")
_sc_params = pltpu.CompilerParams(needs_layout_passes=False)


def _fill(ref, n, value):
    def body(i, _):
        ref[pl.ds(i * LANES, LANES)] = jnp.full((LANES,), value, jnp.float32)
        return 0
    lax.fori_loop(0, n // LANES, body, 0)


@functools.partial(
    pl.kernel,
    mesh=_vmesh,
    out_type=[
        jax.ShapeDtypeStruct((NC * TPAD,), jnp.float32),
        jax.ShapeDtypeStruct((NC * TPAD,), jnp.float32),
    ],
    compiler_params=_sc_params,
    scratch_types=[
        pltpu.VMEM((CHUNK,), jnp.int32),
        pltpu.VMEM((CHUNK,), jnp.int32),
        pltpu.VMEM((CHUNK,), jnp.int32),
        pltpu.VMEM((CHUNK,), jnp.int32),
        pltpu.VMEM((CHUNK,), jnp.float32),
        pltpu.VMEM((CHUNK,), jnp.float32),
        pltpu.VMEM((CHUNK,), jnp.float32),
        pltpu.VMEM((TSLICE,), jnp.float32),
        pltpu.VMEM_SHARED((TPAD,), jnp.float32),
        pltpu.VMEM_SHARED((TPAD,), jnp.float32),
        pltpu.VMEM_SHARED((TPAD,), jnp.float32),
        pltpu.SemaphoreType.DMA,
        pltpu.SemaphoreType.DMA,
        pltpu.SemaphoreType.DMA,
    ],
)
def _edge_accum(x_hbm, he_hbm, s_out, c_out, ni0, ni1, ei0, ei1, v0, v1,
                ones_v, stage_v, x_sp, s_sp, c_sp, gsem, sem0, sem1):
    cid = lax.axis_index("c")
    sid = lax.axis_index("s")
    wid = sid * NC + cid
    tlo = sid * TSLICE
    tsl = pl.ds(tlo, TSLICE)

    _fill(stage_v, TSLICE, 0.0)
    pltpu.sync_copy(stage_v, s_sp.at[tsl])
    pltpu.sync_copy(stage_v, c_sp.at[tsl])
    _fill(ones_v, CHUNK, 1.0)
    pltpu.sync_copy(x_hbm.at[tsl], stage_v)
    pltpu.sync_copy(stage_v, x_sp.at[tsl])
    plsc.subcore_barrier()

    def load_idx(chunk, ni, ei):
        base = wid * PER_W + chunk * CHUNK
        pltpu.sync_copy(he_hbm.at[pl.ds(base, CHUNK)], ni)
        pltpu.sync_copy(he_hbm.at[pl.ds(N_CONN + base, CHUNK)], ei)

    load_idx(0, ni0, ei0)
    pltpu.async_copy(x_sp.at[ni0], v0, gsem).wait()

    def pipe_body(j, _):
        c0 = 2 * j
        s0 = pltpu.async_copy(v0, s_sp.at[ei0], sem0, add=True)
        c0d = pltpu.async_copy(ones_v, c_sp.at[ei0], sem0, add=True)
        load_idx(c0 + 1, ni1, ei1)
        pltpu.async_copy(x_sp.at[ni1], v1, gsem).wait()
        s1 = pltpu.async_copy(v1, s_sp.at[ei1], sem1, add=True)
        c1d = pltpu.async_copy(ones_v, c_sp.at[ei1], sem1, add=True)
        s0.wait()
        c0d.wait()
        load_idx(jnp.minimum(c0 + 2, NCH - 1), ni0, ei0)
        pltpu.async_copy(x_sp.at[ni0], v0, gsem).wait()
        s1.wait()
        c1d.wait()
        return 0

    lax.fori_loop(0, NCH // 2, pipe_body, 0)
    plsc.subcore_barrier()

    pltpu.sync_copy(s_sp.at[tsl], stage_v)
    pltpu.sync_copy(stage_v, s_out.at[pl.ds(cid * TPAD + tlo, TSLICE)])
    pltpu.sync_copy(c_sp.at[tsl], stage_v)
    pltpu.sync_copy(stage_v, c_out.at[pl.ds(cid * TPAD + tlo, TSLICE)])


@functools.partial(
    pl.kernel,
    mesh=_vmesh,
    out_type=jax.ShapeDtypeStruct((NC * TPAD,), jnp.float32),
    compiler_params=_sc_params,
    scratch_types=[
        pltpu.VMEM((CHUNK,), jnp.int32),
        pltpu.VMEM((CHUNK,), jnp.int32),
        pltpu.VMEM((CHUNK,), jnp.int32),
        pltpu.VMEM((CHUNK,), jnp.int32),
        pltpu.VMEM((CHUNK,), jnp.float32),
        pltpu.VMEM((CHUNK,), jnp.float32),
        pltpu.VMEM((TSLICE,), jnp.float32),
        pltpu.VMEM((TSLICE,), jnp.float32),
        pltpu.VMEM((TSLICE,), jnp.float32),
        pltpu.VMEM((TSLICE,), jnp.float32),
        pltpu.VMEM((TSLICE,), jnp.float32),
        pltpu.VMEM((TSLICE,), jnp.float32),
        pltpu.VMEM_SHARED((TPAD,), jnp.float32),
        pltpu.VMEM_SHARED((TPAD,), jnp.float32),
        pltpu.SemaphoreType.DMA,
        pltpu.SemaphoreType.DMA,
        pltpu.SemaphoreType.DMA,
    ],
)
def _node_scatter(he_hbm, s_hbm, c_hbm, wh_hbm, se_hbm, out_p, ni0, ni1,
                  ei0, ei1, v0, v1, sa_v, sb_v, ca_v, cb_v, wh_v, se_v,
                  g_sp, o_sp, gsem, sem0, sem1):
    cid = lax.axis_index("c")
    sid = lax.axis_index("s")
    wid = sid * NC + cid
    tlo = sid * TSLICE
    tsl = pl.ds(tlo, TSLICE)

    pltpu.sync_copy(s_hbm.at[tsl], sa_v)
    pltpu.sync_copy(s_hbm.at[pl.ds(TPAD + tlo, TSLICE)], sb_v)
    pltpu.sync_copy(c_hbm.at[tsl], ca_v)
    pltpu.sync_copy(c_hbm.at[pl.ds(TPAD + tlo, TSLICE)], cb_v)
    pltpu.sync_copy(wh_hbm.at[tsl], wh_v)
    pltpu.sync_copy(se_hbm.at[tsl], se_v)

    def g_body(i, _):
        dsl = pl.ds(i * LANES, LANES)
        s = sa_v[dsl] + sb_v[dsl]
        c = jnp.maximum(ca_v[dsl] + cb_v[dsl], 1.0)
        w = wh_v[dsl] * se_v[dsl]
        se_v[dsl] = w * w * s / c
        return 0

    lax.fori_loop(0, TSLICE // LANES, g_body, 0)
    pltpu.sync_copy(se_v, g_sp.at[tsl])

    _fill(sa_v, TSLICE, 0.0)
    pltpu.sync_copy(sa_v, o_sp.at[tsl])
    plsc.subcore_barrier()

    def load_idx(chunk, ni, ei):
        base = wid * PER_W + chunk * CHUNK
        pltpu.sync_copy(he_hbm.at[pl.ds(base, CHUNK)], ni)
        pltpu.sync_copy(he_hbm.at[pl.ds(N_CONN + base, CHUNK)], ei)

    load_idx(0, ni0, ei0)
    pltpu.async_copy(g_sp.at[ei0], v0, gsem).wait()

    def pipe_body(j, _):
        c0 = 2 * j
        s0 = pltpu.async_copy(v0, o_sp.at[ni0], sem0, add=True)
        load_idx(c0 + 1, ni1, ei1)
        pltpu.async_copy(g_sp.at[ei1], v1, gsem).wait()
        s1 = pltpu.async_copy(v1, o_sp.at[ni1], sem1, add=True)
        s0.wait()
        load_idx(jnp.minimum(c0 + 2, NCH - 1), ni0, ei0)
        pltpu.async_copy(g_sp.at[ei0], v0, gsem).wait()
        s1.wait()
        return 0

    lax.fori_loop(0, NCH // 2, pipe_body, 0)
    plsc.subcore_barrier()

    pltpu.sync_copy(o_sp.at[tsl], sa_v)
    pltpu.sync_copy(sa_v, out_p.at[pl.ds(cid * TPAD + tlo, TSLICE)])


def _combine_body(p_ref, wn_ref, b_ref, o_ref):
    o_ref[...] = ((p_ref[0:1, :] + p_ref[1:2, :])
                  * (W_MAX * W_MAX * wn_ref[0, 0]) + b_ref[0, 0])


def kernel(x_in, hyperedge_index, weight_node, bias, w_hat, s_e):
    x_pad = jnp.zeros((TPAD,), jnp.float32).at[:N_NODES].set(x_in[:, 0])
    wh_pad = jnp.zeros((TPAD,), jnp.float32).at[:N_EDGES].set(w_hat)
    se_pad = jnp.zeros((TPAD,), jnp.float32).at[:N_EDGES].set(s_e)
    he = hyperedge_index.astype(jnp.int32)
    he_flat = jnp.concatenate([he[0], he[1]])

    s_part, c_part = _edge_accum(x_pad, he_flat)
    out_part = _node_scatter(he_flat, s_part, c_part, wh_pad, se_pad)

    out2 = pl.pallas_call(
        _combine_body,
        out_shape=jax.ShapeDtypeStruct((1, TPAD), jnp.float32),
    )(out_part.reshape(NC, TPAD), weight_node, bias.reshape(1, 1))
    return out2[0, :N_NODES, None]


# speedup vs baseline: 1.1479x; 1.1201x over previous
import functools

import jax
import jax.numpy as jnp
from jax import lax
from jax.experimental import pallas as pl
from jax.experimental.pallas import tpu as pltpu
from jax.experimental.pallas import tpu_sc as plsc

N_NODES = 100000
N_EDGES = 100000
N_CONN = 6400000
W_MAX = 1.0

NC = 2
NS = 16
LANES = 16

NW = NC * NS
PER_W = N_CONN // NW
CHUNK = 10000
NCH = PER_W // CHUNK
TPAD = 102400
TSLICE = TPAD // NS

_vmesh = plsc.VectorSubcoreMesh(core_axis_name="c", subcore_axis_name="s")
_sc_params = pltpu.CompilerParams(needs_layout_passes=False)


def _fill(ref, n, value):
    def body(i, _):
        ref[pl.ds(i * LANES, LANES)] = jnp.full((LANES,), value, jnp.float32)
        return 0
    lax.fori_loop(0, n // LANES, body, 0)


@functools.partial(
    pl.kernel,
    mesh=_vmesh,
    out_type=[
        jax.ShapeDtypeStruct((NC * TPAD,), jnp.float32),
        jax.ShapeDtypeStruct((NC * TPAD,), jnp.float32),
    ],
    compiler_params=_sc_params,
    scratch_types=[
        pltpu.VMEM((CHUNK,), jnp.int32),
        pltpu.VMEM((CHUNK,), jnp.int32),
        pltpu.VMEM((CHUNK,), jnp.int32),
        pltpu.VMEM((CHUNK,), jnp.int32),
        pltpu.VMEM((CHUNK,), jnp.float32),
        pltpu.VMEM((CHUNK,), jnp.float32),
        pltpu.VMEM((CHUNK,), jnp.float32),
        pltpu.VMEM((TSLICE,), jnp.float32),
        pltpu.VMEM_SHARED((TPAD,), jnp.float32),
        pltpu.VMEM_SHARED((TPAD,), jnp.float32),
        pltpu.VMEM_SHARED((TPAD,), jnp.float32),
        pltpu.SemaphoreType.DMA,
        pltpu.SemaphoreType.DMA,
        pltpu.SemaphoreType.DMA,
    ],
)
def _edge_accum(x_hbm, he_hbm, s_out, c_out, ni0, ni1, ei0, ei1, v0, v1,
                ones_v, stage_v, x_sp, s_sp, c_sp, gsem, sem0, sem1):
    cid = lax.axis_index("c")
    sid = lax.axis_index("s")
    wid = sid * NC + cid
    tlo = sid * TSLICE
    tsl = pl.ds(tlo, TSLICE)

    _fill(stage_v, TSLICE, 0.0)
    pltpu.sync_copy(stage_v, s_sp.at[tsl])
    pltpu.sync_copy(stage_v, c_sp.at[tsl])
    _fill(ones_v, CHUNK, 1.0)
    pltpu.sync_copy(x_hbm.at[tsl], stage_v)
    pltpu.sync_copy(stage_v, x_sp.at[tsl])
    plsc.subcore_barrier()

    def load_idx(chunk, ni, ei):
        base = wid * PER_W + chunk * CHUNK
        pltpu.sync_copy(he_hbm.at[pl.ds(base, CHUNK)], ni)
        pltpu.sync_copy(he_hbm.at[pl.ds(N_CONN + base, CHUNK)], ei)

    load_idx(0, ni0, ei0)
    pltpu.async_copy(x_sp.at[ni0], v0, gsem).wait()

    def pipe_body(j, _):
        c0 = 2 * j
        s0 = pltpu.async_copy(v0, s_sp.at[ei0], sem0, add=True)
        c0d = pltpu.async_copy(ones_v, c_sp.at[ei0], sem0, add=True)
        load_idx(c0 + 1, ni1, ei1)
        pltpu.async_copy(x_sp.at[ni1], v1, gsem).wait()
        s1 = pltpu.async_copy(v1, s_sp.at[ei1], sem1, add=True)
        c1d = pltpu.async_copy(ones_v, c_sp.at[ei1], sem1, add=True)
        s0.wait()
        c0d.wait()
        load_idx(jnp.minimum(c0 + 2, NCH - 1), ni0, ei0)
        pltpu.async_copy(x_sp.at[ni0], v0, gsem).wait()
        s1.wait()
        c1d.wait()
        return 0

    lax.fori_loop(0, NCH // 2, pipe_body, 0)
    plsc.subcore_barrier()

    pltpu.sync_copy(s_sp.at[tsl], stage_v)
    pltpu.sync_copy(stage_v, s_out.at[pl.ds(cid * TPAD + tlo, TSLICE)])
    pltpu.sync_copy(c_sp.at[tsl], stage_v)
    pltpu.sync_copy(stage_v, c_out.at[pl.ds(cid * TPAD + tlo, TSLICE)])


@functools.partial(
    pl.kernel,
    mesh=_vmesh,
    out_type=jax.ShapeDtypeStruct((NC * TPAD,), jnp.float32),
    compiler_params=_sc_params,
    scratch_types=[
        pltpu.VMEM((CHUNK,), jnp.int32),
        pltpu.VMEM((CHUNK,), jnp.int32),
        pltpu.VMEM((CHUNK,), jnp.int32),
        pltpu.VMEM((CHUNK,), jnp.int32),
        pltpu.VMEM((CHUNK,), jnp.float32),
        pltpu.VMEM((CHUNK,), jnp.float32),
        pltpu.VMEM((TSLICE,), jnp.float32),
        pltpu.VMEM((TSLICE,), jnp.float32),
        pltpu.VMEM((TSLICE,), jnp.float32),
        pltpu.VMEM((TSLICE,), jnp.float32),
        pltpu.VMEM((TSLICE,), jnp.float32),
        pltpu.VMEM((TSLICE,), jnp.float32),
        pltpu.VMEM_SHARED((TPAD,), jnp.float32),
        pltpu.VMEM_SHARED((TPAD,), jnp.float32),
        pltpu.SemaphoreType.DMA,
        pltpu.SemaphoreType.DMA,
        pltpu.SemaphoreType.DMA,
    ],
)
def _node_scatter(he_hbm, s_hbm, c_hbm, wh_hbm, se_hbm, out_p, ni0, ni1,
                  ei0, ei1, v0, v1, sa_v, sb_v, ca_v, cb_v, wh_v, se_v,
                  g_sp, o_sp, gsem, sem0, sem1):
    cid = lax.axis_index("c")
    sid = lax.axis_index("s")
    wid = sid * NC + cid
    tlo = sid * TSLICE
    tsl = pl.ds(tlo, TSLICE)

    pltpu.sync_copy(s_hbm.at[tsl], sa_v)
    pltpu.sync_copy(s_hbm.at[pl.ds(TPAD + tlo, TSLICE)], sb_v)
    pltpu.sync_copy(c_hbm.at[tsl], ca_v)
    pltpu.sync_copy(c_hbm.at[pl.ds(TPAD + tlo, TSLICE)], cb_v)
    pltpu.sync_copy(wh_hbm.at[tsl], wh_v)
    pltpu.sync_copy(se_hbm.at[tsl], se_v)

    def g_body(i, _):
        dsl = pl.ds(i * LANES, LANES)
        s = sa_v[dsl] + sb_v[dsl]
        c = jnp.maximum(ca_v[dsl] + cb_v[dsl], 1.0)
        w = wh_v[dsl] * se_v[dsl]
        se_v[dsl] = w * w * s / c
        return 0

    lax.fori_loop(0, TSLICE // LANES, g_body, 0)
    pltpu.sync_copy(se_v, g_sp.at[tsl])

    _fill(sa_v, TSLICE, 0.0)
    pltpu.sync_copy(sa_v, o_sp.at[tsl])
    plsc.subcore_barrier()

    def load_idx(chunk, ni, ei):
        base = wid * PER_W + chunk * CHUNK
        pltpu.sync_copy(he_hbm.at[pl.ds(base, CHUNK)], ni)
        pltpu.sync_copy(he_hbm.at[pl.ds(N_CONN + base, CHUNK)], ei)

    load_idx(0, ni0, ei0)
    pltpu.async_copy(g_sp.at[ei0], v0, gsem).wait()

    def pipe_body(j, _):
        c0 = 2 * j
        s0 = pltpu.async_copy(v0, o_sp.at[ni0], sem0, add=True)
        load_idx(c0 + 1, ni1, ei1)
        pltpu.async_copy(g_sp.at[ei1], v1, gsem).wait()
        s1 = pltpu.async_copy(v1, o_sp.at[ni1], sem1, add=True)
        s0.wait()
        load_idx(jnp.minimum(c0 + 2, NCH - 1), ni0, ei0)
        pltpu.async_copy(g_sp.at[ei0], v0, gsem).wait()
        s1.wait()
        return 0

    lax.fori_loop(0, NCH // 2, pipe_body, 0)
    plsc.subcore_barrier()

    pltpu.sync_copy(o_sp.at[tsl], sa_v)
    pltpu.sync_copy(sa_v, out_p.at[pl.ds(cid * TPAD + tlo, TSLICE)])


def _combine_body(p_ref, wn_ref, b_ref, o_ref):
    o_ref[...] = ((p_ref[0:1, :] + p_ref[1:2, :])
                  * (W_MAX * W_MAX * wn_ref[0, 0]) + b_ref[0, 0])


def kernel(x_in, hyperedge_index, weight_node, bias, w_hat, s_e):
    x_pad = jnp.zeros((TPAD,), jnp.float32).at[:N_NODES].set(x_in[:, 0])
    wh_pad = jnp.zeros((TPAD,), jnp.float32).at[:N_EDGES].set(w_hat)
    se_pad = jnp.zeros((TPAD,), jnp.float32).at[:N_EDGES].set(s_e)
    he_flat = hyperedge_index.astype(jnp.int32).reshape(2 * N_CONN)

    s_part, c_part = _edge_accum(x_pad, he_flat)
    out_part = _node_scatter(he_flat, s_part, c_part, wh_pad, se_pad)

    out2 = pl.pallas_call(
        _combine_body,
        out_shape=jax.ShapeDtypeStruct((1, TPAD), jnp.float32),
    )(out_part.reshape(NC, TPAD), weight_node, bias.reshape(1, 1))
    return out2[0, :N_NODES, None]
